# Initial kernel scaffold; baseline (speedup 1.0000x reference)
import functools

import jax
import jax.numpy as jnp
from jax import lax
from jax.experimental import pallas as pl
from jax.experimental.pallas import tpu as pltpu
from jax.experimental.pallas import tpu_sc as plsc

_NUM_BOXES = 20000
_NUM_CLASSES = 80
_NUM_SEL = 20000


_BLK = 2048
_GRID = (_NUM_BOXES + _BLK - 1) // _BLK


def _table_body(s_ref, b_ref, t_ref):
    s = s_ref[...]
    bx = b_ref[...]
    m = jnp.max(s, axis=0, keepdims=True)
    ids = lax.broadcasted_iota(jnp.int32, s.shape, 0)
    cl = jnp.min(jnp.where(s == m, ids, _NUM_CLASSES), axis=0, keepdims=True)
    cols = jnp.concatenate(
        [bx, m, cl.astype(jnp.float32), jnp.zeros((2, s.shape[1]), jnp.float32)],
        axis=0,
    )
    eye8 = jnp.eye(8, dtype=jnp.float32)
    t_ref[...] = lax.dot_general(
        cols, eye8, (((0,), (0,)), ((), ())),
        preferred_element_type=jnp.float32,
    )


def _build_table(scores2d, boxes2d):
    return pl.pallas_call(
        _table_body,
        grid=(_GRID,),
        in_specs=[
            pl.BlockSpec((_NUM_CLASSES, _BLK), lambda i: (0, i)),
            pl.BlockSpec((4, _BLK), lambda i: (0, i)),
        ],
        out_specs=pl.BlockSpec((_BLK, 8), lambda i: (i, 0)),
        out_shape=jax.ShapeDtypeStruct((_NUM_BOXES, 8), jnp.float32),
    )(scores2d, boxes2d)


_NUM_SC_CORES = 2
_NUM_SC_SUBCORES = 16
_NW = _NUM_SC_CORES * _NUM_SC_SUBCORES
_CH = 160
_NCHUNK = _NUM_SEL // _CH

_mesh = plsc.VectorSubcoreMesh(
    core_axis_name="c", subcore_axis_name="s",
    num_cores=_NUM_SC_CORES, num_subcores=_NUM_SC_SUBCORES,
)


@functools.partial(
    pl.kernel,
    out_type=jax.ShapeDtypeStruct((_NUM_SEL, 8), jnp.float32),
    mesh=_mesh,
    scratch_types=[
        pltpu.VMEM((_CH,), jnp.int32),
        pltpu.VMEM((_CH, 8), jnp.float32),
        pltpu.SemaphoreType.DMA,
    ],
)
def _sc_gather(table_hbm, idx_hbm, out_hbm, idx_v, rows_v, sem):
    wid = lax.axis_index("s") * _NUM_SC_CORES + lax.axis_index("c")
    rem = _NCHUNK % _NW
    nk = jnp.where(wid < rem, _NCHUNK // _NW + 1, _NCHUNK // _NW)

    def chunk_body(k, carry):
        base = (wid + k * _NW) * _CH
        pltpu.sync_copy(idx_hbm.at[pl.ds(base, _CH)], idx_v)
        pltpu.async_copy(table_hbm.at[idx_v], rows_v, sem).wait()
        pltpu.sync_copy(rows_v, out_hbm.at[pl.ds(base, _CH)])
        return carry

    lax.fori_loop(0, nk, chunk_body, 0)


def kernel(idxTensor, boxes, scores):
    table = _build_table(scores[0], boxes[0])
    idx = idxTensor[:, 2]
    out8 = _sc_gather(table, idx)
    concatenated = out8[:, :6]
    batches = idxTensor[:, 0]
    return (concatenated, batches)


# speedup vs baseline: 4.0840x; 4.0840x over previous
import functools

import jax
import jax.numpy as jnp
from jax import lax
from jax.experimental import pallas as pl
from jax.experimental.pallas import tpu as pltpu
from jax.experimental.pallas import tpu_sc as plsc

_NUM_BOXES = 20000
_NUM_CLASSES = 80
_NUM_SEL = 20000


_BLK = 2048
_GRID = (_NUM_BOXES + _BLK - 1) // _BLK


def _table_body(s_ref, b_ref, t_ref):
    s = s_ref[...]
    bx = b_ref[...]
    m = jnp.max(s, axis=0, keepdims=True)
    ids = lax.broadcasted_iota(jnp.int32, s.shape, 0)
    cl = jnp.min(jnp.where(s == m, ids, _NUM_CLASSES), axis=0, keepdims=True)
    cols = jnp.concatenate(
        [bx, m, cl.astype(jnp.float32), jnp.zeros((2, s.shape[1]), jnp.float32)],
        axis=0,
    )
    eye8 = jnp.eye(8, dtype=jnp.float32)
    t_ref[...] = lax.dot_general(
        cols, eye8, (((0,), (0,)), ((), ())),
        preferred_element_type=jnp.float32,
    )


def _build_table(scores2d, boxes2d):
    return pl.pallas_call(
        _table_body,
        grid=(_GRID,),
        in_specs=[
            pl.BlockSpec((_NUM_CLASSES, _BLK), lambda i: (0, i)),
            pl.BlockSpec((4, _BLK), lambda i: (0, i)),
        ],
        out_specs=pl.BlockSpec((_BLK, 8), lambda i: (i, 0)),
        out_shape=jax.ShapeDtypeStruct((_NUM_BOXES, 8), jnp.float32),
    )(scores2d, boxes2d)


_NUM_SC_CORES = 2
_NUM_SC_SUBCORES = 16
_NW = _NUM_SC_CORES * _NUM_SC_SUBCORES
_CH = 160
_NCHUNK = _NUM_SEL // _CH

_mesh = plsc.VectorSubcoreMesh(
    core_axis_name="c", subcore_axis_name="s",
    num_cores=_NUM_SC_CORES, num_subcores=_NUM_SC_SUBCORES,
)


@functools.partial(
    pl.kernel,
    out_type=jax.ShapeDtypeStruct((_NUM_SEL, 8), jnp.float32),
    mesh=_mesh,
    scratch_types=[
        pltpu.VMEM((_CH,), jnp.int32),
        pltpu.VMEM((_CH, 8), jnp.float32),
        pltpu.SemaphoreType.DMA,
    ],
    compiler_params=pltpu.CompilerParams(use_tc_tiling_on_sc=False),
)
def _sc_gather(table_hbm, idx_hbm, out_hbm, idx_v, rows_v, sem):
    wid = lax.axis_index("s") * _NUM_SC_CORES + lax.axis_index("c")
    rem = _NCHUNK % _NW
    nk = jnp.where(wid < rem, _NCHUNK // _NW + 1, _NCHUNK // _NW)

    def chunk_body(k, carry):
        base = (wid + k * _NW) * _CH
        pltpu.sync_copy(idx_hbm.at[pl.ds(base, _CH)], idx_v)
        pltpu.async_copy(table_hbm.at[idx_v], rows_v, sem).wait()
        pltpu.sync_copy(rows_v, out_hbm.at[pl.ds(base, _CH)])
        return carry

    lax.fori_loop(0, nk, chunk_body, 0)


def kernel(idxTensor, boxes, scores):
    table = _build_table(scores[0], boxes[0])
    idx = idxTensor[:, 2]
    out8 = _sc_gather(table, idx)
    concatenated = out8[:, :6]
    batches = idxTensor[:, 0]
    return (concatenated, batches)


# speedup vs baseline: 4.2774x; 1.0473x over previous
import functools

import jax
import jax.numpy as jnp
from jax import lax
from jax.experimental import pallas as pl
from jax.experimental.pallas import tpu as pltpu
from jax.experimental.pallas import tpu_sc as plsc

_NUM_BOXES = 20000
_NUM_CLASSES = 80
_NUM_SEL = 20000


_BLK = 2048
_GRID = (_NUM_BOXES + _BLK - 1) // _BLK


def _table_body(s_ref, b_ref, t_ref):
    s = s_ref[...]
    bx = b_ref[...]
    m = jnp.max(s, axis=0, keepdims=True)
    ids = lax.broadcasted_iota(jnp.int32, s.shape, 0)
    cl = jnp.min(jnp.where(s == m, ids, _NUM_CLASSES), axis=0, keepdims=True)
    cols = jnp.concatenate(
        [bx, m, cl.astype(jnp.float32), jnp.zeros((2, s.shape[1]), jnp.float32)],
        axis=0,
    )
    t_ref[...] = jnp.transpose(cols)


def _build_table(scores2d, boxes2d):
    return pl.pallas_call(
        _table_body,
        grid=(_GRID,),
        in_specs=[
            pl.BlockSpec((_NUM_CLASSES, _BLK), lambda i: (0, i)),
            pl.BlockSpec((4, _BLK), lambda i: (0, i)),
        ],
        out_specs=pl.BlockSpec((_BLK, 8), lambda i: (i, 0)),
        out_shape=jax.ShapeDtypeStruct((_NUM_BOXES, 8), jnp.float32),
    )(scores2d, boxes2d)


_NUM_SC_CORES = 2
_NUM_SC_SUBCORES = 16
_NW = _NUM_SC_CORES * _NUM_SC_SUBCORES
_CHW = 624
_TAIL_BASE = _NW * _CHW
_TAIL = _NUM_SEL - _TAIL_BASE

_mesh = plsc.VectorSubcoreMesh(
    core_axis_name="c", subcore_axis_name="s",
    num_cores=_NUM_SC_CORES, num_subcores=_NUM_SC_SUBCORES,
)


@functools.partial(
    pl.kernel,
    out_type=jax.ShapeDtypeStruct((_NUM_SEL, 8), jnp.float32),
    mesh=_mesh,
    scratch_types=[
        pltpu.VMEM((_CHW,), jnp.int32),
        pltpu.VMEM((_CHW, 8), jnp.float32),
        pltpu.VMEM((_TAIL,), jnp.int32),
        pltpu.VMEM((_TAIL, 8), jnp.float32),
        pltpu.SemaphoreType.DMA,
    ],
    compiler_params=pltpu.CompilerParams(use_tc_tiling_on_sc=False),
)
def _sc_gather(table_hbm, idx_hbm, out_hbm, idx_v, rows_v, tidx_v, trows_v, sem):
    wid = lax.axis_index("s") * _NUM_SC_CORES + lax.axis_index("c")
    base = wid * _CHW
    pltpu.sync_copy(idx_hbm.at[pl.ds(base, _CHW)], idx_v)
    pltpu.async_copy(table_hbm.at[idx_v], rows_v, sem).wait()
    pltpu.sync_copy(rows_v, out_hbm.at[pl.ds(base, _CHW)])

    @pl.when(wid == 0)
    def _tail():
        pltpu.sync_copy(idx_hbm.at[pl.ds(_TAIL_BASE, _TAIL)], tidx_v)
        pltpu.async_copy(table_hbm.at[tidx_v], trows_v, sem).wait()
        pltpu.sync_copy(trows_v, out_hbm.at[pl.ds(_TAIL_BASE, _TAIL)])


def kernel(idxTensor, boxes, scores):
    table = _build_table(scores[0], boxes[0])
    idx = idxTensor[:, 2]
    out8 = _sc_gather(table, idx)
    concatenated = out8[:, :6]
    batches = idxTensor[:, 0]
    return (concatenated, batches)
